# baseline (device time: 13268 ns/iter reference)
import jax
import jax.numpy as jnp
from jax import lax
from jax.experimental import pallas as pl
from jax.experimental.pallas import tpu as pltpu

N_DEV = 4


def kernel(q, k, v):
    s_per, d = q.shape
    QMAX = 4.0
    DQ = QMAX / 127.0
    LOG2E = 1.4426950408889634
    scale = DQ * LOG2E / (d ** 0.5)

    def body(q_ref, k_ref, v_ref, out_ref, kv, send_sems, recv_sems):
        my = lax.axis_index("i")
        left = lax.rem(my + N_DEV - 1, N_DEV)
        right = lax.rem(my + 1, N_DEV)

        barrier_sem = pltpu.get_barrier_semaphore()
        for nbr in [left, right]:
            pl.semaphore_signal(
                barrier_sem, inc=1,
                device_id=(nbr,), device_id_type=pl.DeviceIdType.MESH,
            )
        kv[0, 0] = jnp.round(jnp.clip(k_ref[...] / DQ, -127, 127)).astype(jnp.int8)
        kv[0, 1] = jnp.round(jnp.clip(v_ref[...] / DQ, -127, 127)).astype(jnp.int8)
        pl.semaphore_wait(barrier_sem, 2)

        def copy(src, dst, sem, dev):
            return pltpu.make_async_remote_copy(
                src_ref=src, dst_ref=dst,
                send_sem=send_sems.at[sem], recv_sem=recv_sems.at[sem],
                device_id=(dev,), device_id_type=pl.DeviceIdType.MESH,
            )

        cp_r = copy(kv.at[0], kv.at[1], 0, right)
        cp_l = copy(kv.at[0], kv.at[2], 1, left)
        cp_k2 = copy(kv.at[1, 0], kv.at[3, 0], 2, right)
        cp_v2 = copy(kv.at[2, 1], kv.at[3, 1], 3, left)

        cp_r.start()
        cp_l.start()

        q_val = (q_ref[...] * scale).astype(jnp.bfloat16)

        def block(state, kj, vj):
            l, acc = state
            s = jnp.dot(q_val, kj.astype(jnp.bfloat16).T,
                        preferred_element_type=jnp.float32)
            p = jnp.exp2(s)
            l_new = l + p.sum(axis=1, keepdims=True)
            acc_new = acc + jnp.dot(
                p.astype(jnp.bfloat16), vj.astype(jnp.bfloat16),
                preferred_element_type=jnp.float32,
            )
            return l_new, acc_new

        s0 = jnp.dot(q_val, kv[0, 0].astype(jnp.bfloat16).T,
                     preferred_element_type=jnp.float32)
        p0 = jnp.exp2(s0)
        l = p0.sum(axis=1, keepdims=True)
        acc = jnp.dot(p0.astype(jnp.bfloat16), kv[0, 1].astype(jnp.bfloat16),
                      preferred_element_type=jnp.float32)
        state = (l, acc)

        cp_r.wait_recv()
        cp_k2.start()
        state = block(state, kv[1, 0], kv[1, 1])
        cp_l.wait_recv()
        cp_v2.start()
        state = block(state, kv[2, 0], kv[2, 1])

        cp_k2.wait_recv()
        cp_v2.wait_recv()
        l, acc = block(state, kv[3, 0], kv[3, 1])

        out_ref[...] = acc * (DQ / l)

        for cp in (cp_r, cp_l, cp_k2, cp_v2):
            cp.wait_send()

    return pl.pallas_call(
        body,
        out_shape=jax.ShapeDtypeStruct((s_per, d), jnp.float32),
        in_specs=[
            pl.BlockSpec(memory_space=pltpu.VMEM),
            pl.BlockSpec(memory_space=pltpu.VMEM),
            pl.BlockSpec(memory_space=pltpu.VMEM),
        ],
        out_specs=pl.BlockSpec(memory_space=pltpu.VMEM),
        scratch_shapes=[
            pltpu.VMEM((N_DEV, 2, s_per, d), jnp.int8),
            pltpu.SemaphoreType.DMA((4,)),
            pltpu.SemaphoreType.DMA((4,)),
        ],
        compiler_params=pltpu.CompilerParams(collective_id=0),
    )(q, k, v)


# device time: 12494 ns/iter; 1.0619x vs baseline; 1.0619x over previous
import jax
import jax.numpy as jnp
from jax import lax
from jax.experimental import pallas as pl
from jax.experimental.pallas import tpu as pltpu

N_DEV = 4


def kernel(q, k, v):
    s_per, d = q.shape
    half = s_per // 2
    QMAX = 4.0
    DQ = QMAX / 127.0
    LOG2E = 1.4426950408889634
    scale = DQ * LOG2E / (d ** 0.5)

    def body(q_ref, k_ref, v_ref, out_ref, kc, vc, send_sems, recv_sems):
        my = lax.axis_index("i")
        left = lax.rem(my + N_DEV - 1, N_DEV)
        right = lax.rem(my + 1, N_DEV)

        barrier_sem = pltpu.get_barrier_semaphore()
        for nbr in [left, right]:
            pl.semaphore_signal(
                barrier_sem, inc=1,
                device_id=(nbr,), device_id_type=pl.DeviceIdType.MESH,
            )
        kc[0] = jnp.round(jnp.clip(k_ref[...] / DQ, -127, 127)).astype(jnp.int8)
        vc[0] = jnp.round(jnp.clip(v_ref[...] / DQ, -127, 127)).astype(jnp.int8)
        pl.semaphore_wait(barrier_sem, 2)

        def copy(src, dst, sem, dev):
            return pltpu.make_async_remote_copy(
                src_ref=src, dst_ref=dst,
                send_sem=send_sems.at[sem], recv_sem=recv_sems.at[sem],
                device_id=(dev,), device_id_type=pl.DeviceIdType.MESH,
            )

        cp_kr = copy(kc.at[0], kc.at[1], 0, right)
        cp_vr = copy(vc.at[0], vc.at[1], 1, right)
        cp_kl = copy(kc.at[0], kc.at[2], 2, left)
        cp_vl = copy(vc.at[0], vc.at[2], 3, left)
        cp_k2a = copy(kc.at[1, pl.ds(0, half)], kc.at[3, pl.ds(0, half)],
                      4, right)
        cp_k2b = copy(kc.at[1, pl.ds(half, half)], kc.at[3, pl.ds(half, half)],
                      5, right)
        cp_v2a = copy(vc.at[2, pl.ds(0, half)], vc.at[3, pl.ds(0, half)],
                      6, left)
        cp_v2b = copy(vc.at[2, pl.ds(half, half)], vc.at[3, pl.ds(half, half)],
                      7, left)

        cp_kr.start()
        cp_vr.start()
        cp_kl.start()
        cp_vl.start()

        q_val = (q_ref[...] * scale).astype(jnp.bfloat16)

        def block(state, kj, vj):
            l, acc = state
            s = jnp.dot(q_val, kj.astype(jnp.bfloat16).T,
                        preferred_element_type=jnp.float32)
            p = jnp.exp2(s)
            l_new = l + p.sum(axis=1, keepdims=True)
            acc_new = acc + jnp.dot(
                p.astype(jnp.bfloat16), vj.astype(jnp.bfloat16),
                preferred_element_type=jnp.float32,
            )
            return l_new, acc_new

        s0 = jnp.dot(q_val, kc[0].astype(jnp.bfloat16).T,
                     preferred_element_type=jnp.float32)
        p0 = jnp.exp2(s0)
        l = p0.sum(axis=1, keepdims=True)
        acc = jnp.dot(p0.astype(jnp.bfloat16), vc[0].astype(jnp.bfloat16),
                      preferred_element_type=jnp.float32)
        state = (l, acc)

        cp_kr.wait_recv()
        cp_k2a.start()
        cp_k2b.start()
        cp_vl.wait_recv()
        cp_v2a.start()
        cp_v2b.start()

        cp_vr.wait_recv()
        state = block(state, kc[1], vc[1])
        cp_kl.wait_recv()
        state = block(state, kc[2], vc[2])

        cp_k2a.wait_recv()
        cp_v2a.wait_recv()
        state = block(state, kc[3, pl.ds(0, half)], vc[3, pl.ds(0, half)])
        cp_k2b.wait_recv()
        cp_v2b.wait_recv()
        l, acc = block(state, kc[3, pl.ds(half, half)],
                       vc[3, pl.ds(half, half)])

        out_ref[...] = acc * (DQ / l)

        for cp in (cp_kr, cp_vr, cp_kl, cp_vl,
                   cp_k2a, cp_k2b, cp_v2a, cp_v2b):
            cp.wait_send()

    return pl.pallas_call(
        body,
        out_shape=jax.ShapeDtypeStruct((s_per, d), jnp.float32),
        in_specs=[
            pl.BlockSpec(memory_space=pltpu.VMEM),
            pl.BlockSpec(memory_space=pltpu.VMEM),
            pl.BlockSpec(memory_space=pltpu.VMEM),
        ],
        out_specs=pl.BlockSpec(memory_space=pltpu.VMEM),
        scratch_shapes=[
            pltpu.VMEM((N_DEV, s_per, d), jnp.int8),
            pltpu.VMEM((N_DEV, s_per, d), jnp.int8),
            pltpu.SemaphoreType.DMA((8,)),
            pltpu.SemaphoreType.DMA((8,)),
        ],
        compiler_params=pltpu.CompilerParams(collective_id=0),
    )(q, k, v)
